# baseline (speedup 1.0000x reference)
import functools
import math

import jax
import jax.numpy as jnp
from jax import lax
from jax.experimental import pallas as pl
from jax.experimental.pallas import tpu as pltpu
from jax.experimental.pallas import tpu_sc as plsc

D_MODEL = 64
SCALE = math.sqrt(D_MODEL)

NUM_CORES = 2
NUM_SUBCORES = 16
NW = NUM_CORES * NUM_SUBCORES

B_TOTAL = 4096 * 200
PER_W = B_TOTAL // NW
CHUNK = 128
NCHUNK = PER_W // CHUNK


def _embed_sc(x_flat, table):
    mesh = plsc.VectorSubcoreMesh(core_axis_name="c", subcore_axis_name="s")

    @functools.partial(
        pl.kernel,
        mesh=mesh,
        out_type=jax.ShapeDtypeStruct((B_TOTAL, D_MODEL), jnp.float32),
        scratch_types=[
            pltpu.VMEM((PER_W,), jnp.int32),
            pltpu.VMEM((CHUNK, D_MODEL), jnp.float32),
            pltpu.SemaphoreType.DMA,
        ],
        compiler_params=pltpu.CompilerParams(use_tc_tiling_on_sc=False),
    )
    def body(x_hbm, tab_hbm, out_hbm, idx_v, rows_v, gsem):
        wid = lax.axis_index("s") * NUM_CORES + lax.axis_index("c")
        base = wid * PER_W
        pltpu.sync_copy(x_hbm.at[pl.ds(base, PER_W)], idx_v)

        def step(ci, _):
            off = ci * CHUNK
            pltpu.async_copy(
                tab_hbm.at[idx_v.at[pl.ds(off, CHUNK)]], rows_v, gsem
            ).wait()

            def srow(i, _):
                for j in range(D_MODEL // 16):
                    sl = pl.ds(j * 16, 16)
                    rows_v[i, sl] = rows_v[i, sl] * SCALE
                return 0

            lax.fori_loop(0, CHUNK, srow, 0)
            pltpu.sync_copy(rows_v, out_hbm.at[pl.ds(base + off, CHUNK)])
            return 0

        lax.fori_loop(0, NCHUNK, step, 0)

    return body(x_flat, table)


def kernel(x, table):
    x_flat = x.reshape(-1)
    out = _embed_sc(x_flat, table)
    return out.reshape(x.shape + (D_MODEL,))


# speedup vs baseline: 1.2042x; 1.2042x over previous
import functools
import math

import jax
import jax.numpy as jnp
from jax import lax
from jax.experimental import pallas as pl
from jax.experimental.pallas import tpu as pltpu
from jax.experimental.pallas import tpu_sc as plsc

D_MODEL = 64
SCALE = math.sqrt(D_MODEL)

NUM_CORES = 2
NUM_SUBCORES = 16
NW = NUM_CORES * NUM_SUBCORES

B_TOTAL = 4096 * 200
PER_W = B_TOTAL // NW
CHUNK = 128
NCHUNK = PER_W // CHUNK
NBUF = 8
PREFETCH = 6
NGROUP = NCHUNK // NBUF
ROW_UNROLL = 4


def _embed_sc(x_flat, table):
    mesh = plsc.VectorSubcoreMesh(core_axis_name="c", subcore_axis_name="s")

    @functools.partial(
        pl.kernel,
        mesh=mesh,
        out_type=jax.ShapeDtypeStruct((B_TOTAL, D_MODEL), jnp.float32),
        scratch_types=[
            pltpu.VMEM((PER_W,), jnp.int32),
            pltpu.VMEM((NBUF, CHUNK, D_MODEL), jnp.float32),
            pltpu.SemaphoreType.DMA((NBUF,)),
            pltpu.SemaphoreType.DMA((NBUF,)),
        ],
        compiler_params=pltpu.CompilerParams(use_tc_tiling_on_sc=False),
    )
    def body(x_hbm, tab_hbm, out_hbm, idx_v, rows_v, gsem, osem):
        wid = lax.axis_index("s") * NUM_CORES + lax.axis_index("c")
        base = wid * PER_W
        pltpu.sync_copy(x_hbm.at[pl.ds(base, PER_W)], idx_v)

        def start_gather(ci, b):
            pltpu.async_copy(
                tab_hbm.at[idx_v.at[pl.ds(ci * CHUNK, CHUNK)]],
                rows_v.at[b], gsem.at[b])

        def wait_gather(ci, b):
            pltpu.make_async_copy(
                tab_hbm.at[idx_v.at[pl.ds(ci * CHUNK, CHUNK)]],
                rows_v.at[b], gsem.at[b]).wait()

        def start_write(ci, b):
            pltpu.async_copy(
                rows_v.at[b], out_hbm.at[pl.ds(base + ci * CHUNK, CHUNK)],
                osem.at[b])

        def wait_write(b):
            pltpu.make_async_copy(
                rows_v.at[b], out_hbm.at[pl.ds(base, CHUNK)],
                osem.at[b]).wait()

        def scale_buf(b):
            def srow(i, _):
                r = i * ROW_UNROLL
                for u in range(ROW_UNROLL):
                    for j in range(D_MODEL // 16):
                        sl = pl.ds(j * 16, 16)
                        rows_v[b, r + u, sl] = rows_v[b, r + u, sl] * SCALE
                return 0
            lax.fori_loop(0, CHUNK // ROW_UNROLL, srow, 0)

        def step(ci, b, prefetch, write_waited):
            wait_gather(ci, b)
            scale_buf(b)
            start_write(ci, b)
            if prefetch:
                bp = (b + PREFETCH) % NBUF
                if write_waited:
                    wait_write(bp)
                start_gather(ci + PREFETCH, bp)

        for ci in range(PREFETCH):
            start_gather(ci, ci)

        for b in range(NBUF):
            step(b, b, prefetch=True, write_waited=(b >= NBUF - PREFETCH))

        def group(g, _):
            for b in range(NBUF):
                step(g * NBUF + b, b, prefetch=True, write_waited=True)
            return 0
        lax.fori_loop(1, NGROUP - 1, group, 0)

        for b in range(NBUF):
            ci = (NGROUP - 1) * NBUF + b
            step(ci, b, prefetch=(ci + PREFETCH < NCHUNK), write_waited=True)

        for b in range(NBUF):
            wait_write(b)

    return body(x_flat, table)


def kernel(x, table):
    x_flat = x.reshape(-1)
    out = _embed_sc(x_flat, table)
    return out.reshape(x.shape + (D_MODEL,))
